# baseline (speedup 1.0000x reference)
import functools

import jax
import jax.numpy as jnp
from jax import lax
from jax.experimental import pallas as pl
from jax.experimental.pallas import tpu as pltpu
from jax.experimental.pallas import tpu_sc as plsc

B = 1024
N = 100000
K = 200
NC = 2
NS = 16
NW = NC * NS
RPW = B // NW
NB = RPW // 8
CW = 6400
TAIL = 32
NSC = N - TAIL
NP = 100096
TOT = B * NP
KP = 256
UCH = B * KP // (NW * 128)

WIDTHS = [CW] * 15 + [NSC - 15 * CW]
OFFSETS = [sum(WIDTHS[:i]) for i in range(len(WIDTHS))]
NCC = len(WIDTHS)
assert OFFSETS[-1] + WIDTHS[-1] == NSC
assert all(o % 128 == 0 and w % 128 == 0 for o, w in zip(OFFSETS, WIDTHS))


def _sc_body(x_hbm, cols_hbm, vals_hbm, y_hbm, flat_v, vals_v, buf0, buf1,
             rowb, sem0, sem1):
    c = lax.axis_index("c")
    s = lax.axis_index("s")
    w = s * NC + c
    r0 = w * RPW

    def x_sl(rb, cc):
        r = r0 + 8 * rb
        return x_hbm.at[pl.ds(r, 8), pl.ds(OFFSETS[cc], WIDTHS[cc])]

    def buf_of(cc):
        full = buf0 if cc % 2 == 0 else buf1
        return full.at[:, pl.ds(0, WIDTHS[cc])]

    def sem_of(cc):
        return sem0 if cc % 2 == 0 else sem1

    pltpu.async_copy(x_sl(0, 0), buf_of(0), sem0)
    pltpu.sync_copy(cols_hbm.at[w], flat_v)
    pltpu.sync_copy(vals_hbm.at[w], vals_v)

    lane = lax.iota(jnp.int32, 16)

    def fl_body(g, carry):
        rowg = r0 + g // 2
        base = rowg * NP
        for t in range(8):
            c16 = flat_v[g, t * 16:(t + 1) * 16]
            flat_v[g, t * 16:(t + 1) * 16] = jnp.where(
                c16 < N, c16 + base, TOT + lane)
        return carry

    lax.fori_loop(0, UCH, fl_body, 0)

    def band(rb, carry):
        for cc in range(NCC):
            pltpu.make_async_copy(x_sl(rb, cc), buf_of(cc), sem_of(cc)).wait()
            if cc + 1 < NCC:
                pltpu.async_copy(x_sl(rb, cc + 1), buf_of(cc + 1),
                                 sem_of(cc + 1))
            else:
                @pl.when(rb < NB - 1)
                def _():
                    pltpu.async_copy(x_sl(rb + 1, 0), buf_of(0), sem_of(0))
            full = buf0 if cc % 2 == 0 else buf1
            for k in range(8):
                fbase = pl.multiple_of(
                    (r0 + 8 * rb + k) * NP + OFFSETS[cc], 128)
                pltpu.sync_copy(full.at[k, pl.ds(0, WIDTHS[cc])],
                                rowb.at[s, 0].at[pl.ds(0, WIDTHS[cc])])
                pltpu.sync_copy(rowb.at[s, 0].at[pl.ds(0, WIDTHS[cc])],
                                y_hbm.at[pl.ds(fbase, WIDTHS[cc])])
        return carry

    lax.fori_loop(0, NB, band, 0)

    def sc_body(g, carry):
        pltpu.sync_copy(vals_v.at[g], y_hbm.at[flat_v.at[g]])
        return carry

    lax.fori_loop(0, UCH, sc_body, 0)


@jax.jit
def _scatter_op(x, cols_3d, vals_3d):
    mesh = plsc.VectorSubcoreMesh(core_axis_name="c", subcore_axis_name="s")
    run = pl.kernel(
        _sc_body,
        out_type=jax.ShapeDtypeStruct((TOT + 128,), jnp.float32),
        mesh=mesh,
        scratch_types=[
            pltpu.VMEM((UCH, 128), jnp.int32),
            pltpu.VMEM((UCH, 128), jnp.float32),
            pltpu.VMEM((8, CW), jnp.float32),
            pltpu.VMEM((8, CW), jnp.float32),
            pltpu.VMEM_SHARED((NS, 1, CW), jnp.float32),
            pltpu.SemaphoreType.DMA,
            pltpu.SemaphoreType.DMA,
        ],
    )
    return run(x, cols_3d, vals_3d)


def kernel(x, indices, src):
    idx = indices.astype(jnp.int32)
    srcf = src.astype(jnp.float32)

    tail_cols = jnp.broadcast_to(
        jnp.arange(NSC, N, dtype=jnp.int32)[None, :], (B, TAIL))
    cols = jnp.concatenate(
        [tail_cols, idx,
         jnp.full((B, KP - TAIL - K), N, dtype=jnp.int32)], axis=1)
    vals = jnp.concatenate(
        [x[:, NSC:], srcf,
         jnp.zeros((B, KP - TAIL - K), dtype=jnp.float32)], axis=1)

    cols_3d = jnp.reshape(cols, (NW, NW * 2, 128))
    vals_3d = jnp.reshape(vals, (NW, NW * 2, 128))
    y = _scatter_op(x, cols_3d, vals_3d)
    return jnp.reshape(y[:TOT], (B, NP))[:, :N]


# speedup vs baseline: 2.4775x; 2.4775x over previous
import functools

import jax
import jax.numpy as jnp
from jax import lax
from jax.experimental import pallas as pl
from jax.experimental.pallas import tpu as pltpu
from jax.experimental.pallas import tpu_sc as plsc

B = 1024
N = 100000
K = 200
NC = 2
NS = 16
NW = NC * NS
TOT = B * N
ELEMS_PER_W = TOT // NW
CP = 32_000
NCH = ELEMS_PER_W // CP
CHUNK = 128
NCHUNK = (B * K) // CHUNK
CH_PER_W = NCHUNK // NW
SUB = CHUNK // 16


def _scatter_body(x_hbm, idx_hbm, src_hbm, out_hbm,
                  idx_v, src_v, flat_v, shared, sem0, sem1):
    c = lax.axis_index("c")
    s = lax.axis_index("s")
    w = s * NC + c
    base = w * ELEMS_PER_W
    buf0 = shared.at[s, 0]
    buf1 = shared.at[s, 1]

    def in_sl(i):
        return x_hbm.at[pl.ds(base + i * CP, CP)]

    def out_sl(i):
        return out_hbm.at[pl.ds(base + i * CP, CP)]

    pltpu.async_copy(in_sl(0), buf0, sem0)

    pltpu.sync_copy(idx_hbm.at[w], idx_v)
    pltpu.sync_copy(src_hbm.at[w], src_v)

    cb = w * CH_PER_W

    def idx_body(j, carry):
        for k in range(SUB):
            t0 = (cb + j) * CHUNK + k * 16
            r0 = t0 // K
            rem = t0 - r0 * K
            lane = lax.iota(jnp.int32, 16)
            bump = jnp.where(lane + rem >= K, jnp.int32(N), jnp.int32(0))
            flat_v[j, k * 16:(k + 1) * 16] = (
                idx_v[j, k * 16:(k + 1) * 16] + (r0 * N + bump))
        return carry

    lax.fori_loop(0, CH_PER_W, idx_body, 0)

    def copy_body(it, carry):
        i0 = 2 * it
        pltpu.make_async_copy(in_sl(i0), buf0, sem0).wait()
        pltpu.async_copy(in_sl(i0 + 1), buf1, sem1)
        pltpu.sync_copy(buf0, out_sl(i0))
        pltpu.make_async_copy(in_sl(i0 + 1), buf1, sem1).wait()

        @pl.when(it < NCH // 2 - 1)
        def _():
            pltpu.async_copy(in_sl(i0 + 2), buf0, sem0)

        pltpu.sync_copy(buf1, out_sl(i0 + 1))
        return carry

    lax.fori_loop(0, NCH // 2, copy_body, 0)

    def sc_body(j, carry):
        pltpu.sync_copy(src_v.at[j], out_hbm.at[flat_v.at[j]])
        return carry

    lax.fori_loop(0, CH_PER_W, sc_body, 0)


@jax.jit
def _scatter_op(x_flat, idx_2d, src_2d):
    mesh = plsc.VectorSubcoreMesh(core_axis_name="c", subcore_axis_name="s")
    run = pl.kernel(
        _scatter_body,
        out_type=jax.ShapeDtypeStruct((TOT,), jnp.float32),
        mesh=mesh,
        scratch_types=[
            pltpu.VMEM((CH_PER_W, CHUNK), jnp.int32),
            pltpu.VMEM((CH_PER_W, CHUNK), jnp.float32),
            pltpu.VMEM((CH_PER_W, CHUNK), jnp.int32),
            pltpu.VMEM_SHARED((NS, 2, CP), jnp.float32),
            pltpu.SemaphoreType.DMA,
            pltpu.SemaphoreType.DMA,
        ],
    )
    return run(x_flat, idx_2d, src_2d)


def kernel(x, indices, src):
    x_flat = jnp.reshape(x, (TOT,))
    idx_2d = jnp.reshape(indices.astype(jnp.int32), (NW, CH_PER_W, CHUNK))
    src_2d = jnp.reshape(src.astype(jnp.float32), (NW, CH_PER_W, CHUNK))
    out_flat = _scatter_op(x_flat, idx_2d, src_2d)
    return jnp.reshape(out_flat, (B, N))


# speedup vs baseline: 2.4797x; 1.0009x over previous
import functools

import jax
import jax.numpy as jnp
from jax import lax
from jax.experimental import pallas as pl
from jax.experimental.pallas import tpu as pltpu
from jax.experimental.pallas import tpu_sc as plsc

B = 1024
N = 100000
K = 200
NC = 2
NS = 16
NW = NC * NS
TOT = B * N
ELEMS_PER_W = TOT // NW
CP = 32_000
NCH = ELEMS_PER_W // CP
CHUNK = 128
NCHUNK = (B * K) // CHUNK
CH_PER_W = NCHUNK // NW
SUB = CHUNK // 16


def _scatter_body(x_hbm, idx_hbm, src_hbm, out_hbm,
                  idx_v, src_v, flat_v, shared, sem0, sem1, semo0, semo1):
    c = lax.axis_index("c")
    s = lax.axis_index("s")
    w = s * NC + c
    base = w * ELEMS_PER_W
    buf0 = shared.at[s, 0]
    buf1 = shared.at[s, 1]

    def in_sl(i):
        return x_hbm.at[pl.ds(base + i * CP, CP)]

    def out_sl(i):
        return out_hbm.at[pl.ds(base + i * CP, CP)]

    pltpu.async_copy(in_sl(0), buf0, sem0)

    pltpu.sync_copy(idx_hbm.at[w], idx_v)
    pltpu.sync_copy(src_hbm.at[w], src_v)

    cb = w * CH_PER_W

    def idx_body(j, carry):
        for k in range(SUB):
            t0 = (cb + j) * CHUNK + k * 16
            r0 = t0 // K
            rem = t0 - r0 * K
            lane = lax.iota(jnp.int32, 16)
            bump = jnp.where(lane + rem >= K, jnp.int32(N), jnp.int32(0))
            flat_v[j, k * 16:(k + 1) * 16] = (
                idx_v[j, k * 16:(k + 1) * 16] + (r0 * N + bump))
        return carry

    lax.fori_loop(0, CH_PER_W, idx_body, 0)

    def copy_body(it, carry):
        i0 = 2 * it
        pltpu.make_async_copy(in_sl(i0), buf0, sem0).wait()
        pltpu.async_copy(buf0, out_sl(i0), semo0)

        @pl.when(it > 0)
        def _():
            pltpu.make_async_copy(buf1, out_sl(i0 - 1), semo1).wait()

        pltpu.async_copy(in_sl(i0 + 1), buf1, sem1)
        pltpu.make_async_copy(in_sl(i0 + 1), buf1, sem1).wait()
        pltpu.async_copy(buf1, out_sl(i0 + 1), semo1)
        pltpu.make_async_copy(buf0, out_sl(i0), semo0).wait()

        @pl.when(it < NCH // 2 - 1)
        def _():
            pltpu.async_copy(in_sl(i0 + 2), buf0, sem0)

        return carry

    lax.fori_loop(0, NCH // 2, copy_body, 0)
    pltpu.make_async_copy(buf1, out_sl(NCH - 1), semo1).wait()

    def sc_body(j, carry):
        pltpu.sync_copy(src_v.at[j], out_hbm.at[flat_v.at[j]])
        return carry

    lax.fori_loop(0, CH_PER_W, sc_body, 0)


@jax.jit
def _scatter_op(x_flat, idx_2d, src_2d):
    mesh = plsc.VectorSubcoreMesh(core_axis_name="c", subcore_axis_name="s")
    run = pl.kernel(
        _scatter_body,
        out_type=jax.ShapeDtypeStruct((TOT,), jnp.float32),
        mesh=mesh,
        scratch_types=[
            pltpu.VMEM((CH_PER_W, CHUNK), jnp.int32),
            pltpu.VMEM((CH_PER_W, CHUNK), jnp.float32),
            pltpu.VMEM((CH_PER_W, CHUNK), jnp.int32),
            pltpu.VMEM_SHARED((NS, 2, CP), jnp.float32),
            pltpu.SemaphoreType.DMA,
            pltpu.SemaphoreType.DMA,
            pltpu.SemaphoreType.DMA,
            pltpu.SemaphoreType.DMA,
        ],
    )
    return run(x_flat, idx_2d, src_2d)


def kernel(x, indices, src):
    x_flat = jnp.reshape(x, (TOT,))
    idx_2d = jnp.reshape(indices.astype(jnp.int32), (NW, CH_PER_W, CHUNK))
    src_2d = jnp.reshape(src.astype(jnp.float32), (NW, CH_PER_W, CHUNK))
    out_flat = _scatter_op(x_flat, idx_2d, src_2d)
    return jnp.reshape(out_flat, (B, N))
